# Initial kernel scaffold; baseline (speedup 1.0000x reference)
import jax
import jax.numpy as jnp
from jax.experimental import pallas as pl


def kernel(x, edge_index, W, b):
    raise NotImplementedError("write your pallas kernel here")


# speedup vs baseline: 5.9004x; 5.9004x over previous
import functools

import jax
import jax.numpy as jnp
from jax import lax
from jax.experimental import pallas as pl
from jax.experimental.pallas import tpu as pltpu
from jax.experimental.pallas import tpu_sc as plsc

N_NODES = 10000
NP = 10240
E = 320000
D = 128
CH = 128
NCH = E // CH
NC, NS = 2, 16
NW = NC * NS
CPW = NCH // NW
CREM = NCH - CPW * NW
RPT = NP // NS

_MESH = plsc.VectorSubcoreMesh(core_axis_name="c", subcore_axis_name="s")


def _worker_id():
    cid = lax.axis_index("c")
    sid = lax.axis_index("s")
    return cid, sid, sid * NC + cid


def _degree_body(sidx_hbm, ridx_hbm, out_hbm, sh, rh, sbuf, rbuf):
    _, _, wid = _worker_id()
    start = wid * CPW + jnp.minimum(wid, CREM)
    cnt = CPW + jnp.where(wid < CREM, 1, 0)

    def zero_body(i, _):
        sh[pl.ds(i * 16, 16)] = jnp.zeros((16,), jnp.float32)
        rh[pl.ds(i * 16, 16)] = jnp.zeros((16,), jnp.float32)
        return 0

    lax.fori_loop(0, NP // 16, zero_body, 0)

    ones = jnp.ones((16,), jnp.float32)

    def chunk_body(j, _):
        @pl.when(j < cnt)
        def _():
            ci = start + j
            pltpu.sync_copy(sidx_hbm.at[ci], sbuf)
            pltpu.sync_copy(ridx_hbm.at[ci], rbuf)
            for k in range(CH // 16):
                plsc.addupdate_scatter(sh, [sbuf[pl.ds(k * 16, 16)]], ones)
                plsc.addupdate_scatter(rh, [rbuf[pl.ds(k * 16, 16)]], ones)
        return 0

    lax.fori_loop(0, CPW + 1, chunk_body, 0)

    pltpu.sync_copy(sh, out_hbm.at[wid, 0])
    pltpu.sync_copy(rh, out_hbm.at[wid, 1])


_SC_PARAMS = pltpu.CompilerParams(needs_layout_passes=False)

_degree_call = pl.kernel(
    _degree_body,
    out_type=jax.ShapeDtypeStruct((NW, 2, NP), jnp.float32),
    mesh=_MESH,
    compiler_params=_SC_PARAMS,
    scratch_types=[
        pltpu.VMEM((NP,), jnp.float32),
        pltpu.VMEM((NP,), jnp.float32),
        pltpu.VMEM((CH,), jnp.int32),
        pltpu.VMEM((CH,), jnp.int32),
    ],
)


BM = 1024


def _mm_body(x_ref, w_ref, b_ref, degs_ref, out_ref):
    nodes = jnp.dot(x_ref[...], w_ref[...],
                    preferred_element_type=jnp.float32) + b_ref[...]
    sdeg = jnp.sum(degs_ref[:, 0, :], axis=0)
    sinv = lax.rsqrt(jnp.maximum(sdeg, 1.0))
    out_ref[...] = nodes * sinv[:, None]


def _mm_call(xp, W, b, degs):
    return pl.pallas_call(
        _mm_body,
        grid=(NP // BM,),
        in_specs=[
            pl.BlockSpec((BM, D), lambda m: (m, 0)),
            pl.BlockSpec((D, D), lambda m: (0, 0)),
            pl.BlockSpec((D,), lambda m: (0,)),
            pl.BlockSpec((NW, 2, BM), lambda m: (0, 0, m)),
        ],
        out_specs=pl.BlockSpec((BM, D), lambda m: (m, 0)),
        out_shape=jax.ShapeDtypeStruct((NP, D), jnp.float32),
    )(xp, W, b, degs)


def _agg_body(table_hbm, sidx_hbm, ridx_hbm, zeros_hbm, out_hbm,
              sidx_v, ridx_v, rows_v, sem, acc):
    cid, sid, wid = _worker_id()
    start = wid * CPW + jnp.minimum(wid, CREM)
    cnt = CPW + jnp.where(wid < CREM, 1, 0)

    pltpu.sync_copy(zeros_hbm.at[pl.ds(sid * RPT, RPT)],
                    acc.at[pl.ds(sid * RPT, RPT)])
    plsc.subcore_barrier()

    def chunk_body(j, _):
        @pl.when(j < cnt)
        def _():
            ci = start + j
            pltpu.sync_copy(sidx_hbm.at[ci], sidx_v)
            pltpu.sync_copy(ridx_hbm.at[ci], ridx_v)
            pltpu.async_copy(table_hbm.at[sidx_v], rows_v, sem).wait()
            pltpu.sync_copy(rows_v, acc.at[ridx_v], add=True)
        return 0

    lax.fori_loop(0, CPW + 1, chunk_body, 0)
    plsc.subcore_barrier()

    pltpu.sync_copy(acc.at[pl.ds(sid * RPT, RPT)],
                    out_hbm.at[cid, pl.ds(sid * RPT, RPT)])


_agg_call = pl.kernel(
    _agg_body,
    out_type=jax.ShapeDtypeStruct((NC, NP, D), jnp.float32),
    mesh=_MESH,
    compiler_params=_SC_PARAMS,
    scratch_types=[
        pltpu.VMEM((CH,), jnp.int32),
        pltpu.VMEM((CH,), jnp.int32),
        pltpu.VMEM((CH, D), jnp.float32),
        pltpu.SemaphoreType.DMA,
        pltpu.VMEM_SHARED((NP, D), jnp.float32),
    ],
)


def _fin_body(p_ref, degs_ref, o_ref):
    s = p_ref[0] + p_ref[1]
    rdeg = jnp.sum(degs_ref[:, 1, :], axis=0)
    rinv = lax.rsqrt(jnp.maximum(rdeg, 1.0))
    o_ref[...] = s * rinv[:, None]


def _fin_call(partial, degs):
    return pl.pallas_call(
        _fin_body,
        grid=(NP // BM,),
        in_specs=[
            pl.BlockSpec((NC, BM, D), lambda m: (0, m, 0)),
            pl.BlockSpec((NW, 2, BM), lambda m: (0, 0, m)),
        ],
        out_specs=pl.BlockSpec((BM, D), lambda m: (m, 0)),
        out_shape=jax.ShapeDtypeStruct((NP, D), jnp.float32),
    )(partial, degs)


def kernel(x, edge_index, W, b):
    senders = edge_index[0].reshape(NCH, CH)
    receivers = edge_index[1].reshape(NCH, CH)

    degs = _degree_call(senders, receivers)

    xp = jnp.pad(x, ((0, NP - N_NODES), (0, 0)))
    table = _mm_call(xp, W, b, degs)

    zeros = jnp.zeros((NP, D), jnp.float32)
    partial = _agg_call(table, senders, receivers, zeros)

    out = _fin_call(partial, degs)
    return out[:N_NODES]
